# baseline (device time: 23369 ns/iter reference)
import jax
import jax.numpy as jnp
from jax import lax
from jax.experimental import pallas as pl
from jax.experimental.pallas import tpu as pltpu

N_DEV = 4
M_PER = 512
HM = M_PER // 2
D = 512
DH = D // 2


def kernel(partial, gamma):
    x = partial.reshape(N_DEV * M_PER, D)
    g = gamma.reshape(1, D)
    x = pltpu.with_memory_space_constraint(x, pltpu.MemorySpace.HBM)
    g = pltpu.with_memory_space_constraint(g, pltpu.MemorySpace.HBM)

    def body(x_ref, g_ref, out_ref,
             recv_a1, recv_b1, send_a2, send_b2, recv_a2, recv_b2,
             vx_own, vx_fa, vx_fb, vg, vout,
             sems_send_a, sems_recv_a, sems_send_b, sems_recv_b,
             sems_local):
        my = lax.axis_index("i")
        left = lax.rem(my + N_DEV - 1, N_DEV)
        right = lax.rem(my + 1, N_DEV)
        q = my ^ 1
        r = 3 - my

        def row(c, h=None):
            if h is None:
                return pl.ds(c * M_PER, M_PER)
            return pl.ds(c * M_PER + h * HM, HM)

        rows0 = pl.ds(0, HM)
        rows1 = pl.ds(HM, HM)
        cols_a = pl.ds(0, DH)
        cols_b = pl.ds(DH, DH)

        cp_own = pltpu.make_async_copy(
            x_ref.at[row(my), :], vx_own, sems_local.at[0])
        cp_fa = pltpu.make_async_copy(
            x_ref.at[row(3 - my), cols_a], vx_fa, sems_local.at[1])
        cp_fb = pltpu.make_async_copy(
            x_ref.at[row(q), cols_b], vx_fb, sems_local.at[2])
        cp_g = pltpu.make_async_copy(g_ref, vg, sems_local.at[3])
        cp_fa.start()
        cp_fb.start()
        cp_own.start()
        cp_g.start()

        barrier_sem = pltpu.get_barrier_semaphore()
        for nbr in (left, right):
            pl.semaphore_signal(
                barrier_sem, inc=1,
                device_id=(nbr,), device_id_type=pl.DeviceIdType.MESH,
            )
        pl.semaphore_wait(barrier_sem, 2)

        def remote(src, dst, ssem, rsem, dev):
            return pltpu.make_async_remote_copy(
                src_ref=src, dst_ref=dst, send_sem=ssem, recv_sem=rsem,
                device_id=(dev,), device_id_type=pl.DeviceIdType.MESH,
            )

        a1_0a = remote(x_ref.at[row(3 - q, 0), cols_a],
                       recv_a1.at[0, rows0], sems_send_a.at[0],
                       sems_recv_a.at[0], q)
        a1_0b = remote(x_ref.at[row(3 - q, 1), cols_a],
                       recv_a1.at[0, rows1], sems_send_a.at[1],
                       sems_recv_a.at[1], q)
        a1_1 = remote(x_ref.at[row(q), cols_a],
                      recv_a1.at[1], sems_send_a.at[2],
                      sems_recv_a.at[2], q)
        b1_0a = remote(x_ref.at[row(r ^ 1, 0), cols_b],
                       recv_b1.at[0, rows0], sems_send_b.at[0],
                       sems_recv_b.at[0], r)
        b1_0b = remote(x_ref.at[row(r ^ 1, 1), cols_b],
                       recv_b1.at[0, rows1], sems_send_b.at[1],
                       sems_recv_b.at[1], r)
        b1_1 = remote(x_ref.at[row(r), cols_b],
                      recv_b1.at[1], sems_send_b.at[2],
                      sems_recv_b.at[2], r)
        a1_0a.start()
        b1_0a.start()
        a1_0b.start()
        b1_0b.start()
        a1_1.start()
        b1_1.start()

        a2a = remote(send_a2.at[rows0], recv_a2.at[rows0],
                     sems_send_a.at[3], sems_recv_a.at[3], r)
        a2b = remote(send_a2.at[rows1], recv_a2.at[rows1],
                     sems_send_a.at[4], sems_recv_a.at[4], r)
        b2a = remote(send_b2.at[rows0], recv_b2.at[rows0],
                     sems_send_b.at[3], sems_recv_b.at[3], q)
        b2b = remote(send_b2.at[rows1], recv_b2.at[rows1],
                     sems_send_b.at[4], sems_recv_b.at[4], q)

        a1_0a.wait_recv()
        cp_fa.wait()
        send_a2[rows0, :] = recv_a1[0, rows0] + vx_fa[rows0, :]
        a2a.start()
        b1_0a.wait_recv()
        cp_fb.wait()
        send_b2[rows0, :] = recv_b1[0, rows0] + vx_fb[rows0, :]
        b2a.start()
        a1_0b.wait_recv()
        send_a2[rows1, :] = recv_a1[0, rows1] + vx_fa[rows1, :]
        a2b.start()
        b1_0b.wait_recv()
        send_b2[rows1, :] = recv_b1[0, rows1] + vx_fb[rows1, :]
        b2b.start()

        a1_1.wait_recv()
        cp_own.wait()
        recv_a1[0, :, :] = recv_a1[1] + vx_own[:, cols_a]
        b1_1.wait_recv()
        recv_b1[0, :, :] = recv_b1[1] + vx_own[:, cols_b]
        cp_g.wait()

        cp_out = [None, None]
        for h, rws in enumerate((rows0, rows1)):
            (a2a, a2b)[h].wait_recv()
            y_a = recv_a1[0, rws] + recv_a2[rws, :]
            (b2a, b2b)[h].wait_recv()
            y_b = recv_b1[0, rws] + recv_b2[rws, :]
            ssq = (jnp.sum(y_a * y_a, axis=-1, keepdims=True)
                   + jnp.sum(y_b * y_b, axis=-1, keepdims=True))
            scale = lax.rsqrt(ssq / D + 1e-6)
            vout[rws, cols_a] = y_a * scale * vg[:, cols_a]
            vout[rws, cols_b] = y_b * scale * vg[:, cols_b]
            cp_out[h] = pltpu.make_async_copy(
                vout.at[rws], out_ref.at[rws], sems_local.at[4 + h])
            cp_out[h].start()
        cp_out[0].wait()
        cp_out[1].wait()

        for d in (a1_0a, a1_0b, a1_1, b1_0a, b1_0b, b1_1,
                  a2a, a2b, b2a, b2b):
            d.wait_send()

    return pl.pallas_call(
        body,
        out_shape=jax.ShapeDtypeStruct((M_PER, D), jnp.float32),
        in_specs=[
            pl.BlockSpec(memory_space=pl.ANY),
            pl.BlockSpec(memory_space=pl.ANY),
        ],
        out_specs=pl.BlockSpec(memory_space=pl.ANY),
        scratch_shapes=[
            pltpu.VMEM((2, M_PER, DH), jnp.float32),
            pltpu.VMEM((2, M_PER, DH), jnp.float32),
            pltpu.VMEM((M_PER, DH), jnp.float32),
            pltpu.VMEM((M_PER, DH), jnp.float32),
            pltpu.VMEM((M_PER, DH), jnp.float32),
            pltpu.VMEM((M_PER, DH), jnp.float32),
            pltpu.VMEM((M_PER, D), jnp.float32),
            pltpu.VMEM((M_PER, DH), jnp.float32),
            pltpu.VMEM((M_PER, DH), jnp.float32),
            pltpu.VMEM((1, D), jnp.float32),
            pltpu.VMEM((M_PER, D), jnp.float32),
            pltpu.SemaphoreType.DMA((5,)),
            pltpu.SemaphoreType.DMA((5,)),
            pltpu.SemaphoreType.DMA((5,)),
            pltpu.SemaphoreType.DMA((5,)),
            pltpu.SemaphoreType.DMA((6,)),
        ],
        compiler_params=pltpu.CompilerParams(collective_id=0),
    )(x, g)


# device time: 23315 ns/iter; 1.0023x vs baseline; 1.0023x over previous
import jax
import jax.numpy as jnp
from jax import lax
from jax.experimental import pallas as pl
from jax.experimental.pallas import tpu as pltpu

N_DEV = 4
M_PER = 512
HM = M_PER // 2
D = 512
DH = D // 2


def kernel(partial, gamma):
    x = partial.reshape(N_DEV * M_PER, D)
    g = gamma.reshape(1, D)
    x = pltpu.with_memory_space_constraint(x, pltpu.MemorySpace.HBM)
    g = pltpu.with_memory_space_constraint(g, pltpu.MemorySpace.HBM)

    def body(x_ref, g_ref, out_ref,
             recv_a1, recv_b1, send_a2, send_b2, recv_a2, recv_b2,
             vx_own, vx_fa, vx_fb, vg,
             sems_send_a, sems_recv_a, sems_send_b, sems_recv_b,
             sems_local):
        my = lax.axis_index("i")
        left = lax.rem(my + N_DEV - 1, N_DEV)
        right = lax.rem(my + 1, N_DEV)
        q = my ^ 1
        r = 3 - my

        def row(c, h=None):
            if h is None:
                return pl.ds(c * M_PER, M_PER)
            return pl.ds(c * M_PER + h * HM, HM)

        rows0 = pl.ds(0, HM)
        rows1 = pl.ds(HM, HM)
        cols_a = pl.ds(0, DH)
        cols_b = pl.ds(DH, DH)

        cp_own = pltpu.make_async_copy(
            x_ref.at[row(my), :], vx_own, sems_local.at[0])
        cp_fa = pltpu.make_async_copy(
            x_ref.at[row(3 - my), cols_a], vx_fa, sems_local.at[1])
        cp_fb = pltpu.make_async_copy(
            x_ref.at[row(q), cols_b], vx_fb, sems_local.at[2])
        cp_g = pltpu.make_async_copy(g_ref, vg, sems_local.at[3])
        cp_fa.start()
        cp_fb.start()
        cp_own.start()
        cp_g.start()

        barrier_sem = pltpu.get_barrier_semaphore()
        for nbr in (left, right):
            pl.semaphore_signal(
                barrier_sem, inc=1,
                device_id=(nbr,), device_id_type=pl.DeviceIdType.MESH,
            )
        pl.semaphore_wait(barrier_sem, 2)

        def remote(src, dst, ssem, rsem, dev):
            return pltpu.make_async_remote_copy(
                src_ref=src, dst_ref=dst, send_sem=ssem, recv_sem=rsem,
                device_id=(dev,), device_id_type=pl.DeviceIdType.MESH,
            )

        a1_0a = remote(x_ref.at[row(3 - q, 0), cols_a],
                       recv_a1.at[0, rows0], sems_send_a.at[0],
                       sems_recv_a.at[0], q)
        a1_0b = remote(x_ref.at[row(3 - q, 1), cols_a],
                       recv_a1.at[0, rows1], sems_send_a.at[1],
                       sems_recv_a.at[1], q)
        a1_1 = remote(x_ref.at[row(q), cols_a],
                      recv_a1.at[1], sems_send_a.at[2],
                      sems_recv_a.at[2], q)
        b1_0a = remote(x_ref.at[row(r ^ 1, 0), cols_b],
                       recv_b1.at[0, rows0], sems_send_b.at[0],
                       sems_recv_b.at[0], r)
        b1_0b = remote(x_ref.at[row(r ^ 1, 1), cols_b],
                       recv_b1.at[0, rows1], sems_send_b.at[1],
                       sems_recv_b.at[1], r)
        b1_1 = remote(x_ref.at[row(r), cols_b],
                      recv_b1.at[1], sems_send_b.at[2],
                      sems_recv_b.at[2], r)
        a1_0a.start()
        b1_0a.start()
        a1_0b.start()
        b1_0b.start()
        a1_1.start()
        b1_1.start()

        a2a = remote(send_a2.at[rows0], recv_a2.at[rows0],
                     sems_send_a.at[3], sems_recv_a.at[3], r)
        a2b = remote(send_a2.at[rows1], recv_a2.at[rows1],
                     sems_send_a.at[4], sems_recv_a.at[4], r)
        b2a = remote(send_b2.at[rows0], recv_b2.at[rows0],
                     sems_send_b.at[3], sems_recv_b.at[3], q)
        b2b = remote(send_b2.at[rows1], recv_b2.at[rows1],
                     sems_send_b.at[4], sems_recv_b.at[4], q)

        a1_0a.wait_recv()
        cp_fa.wait()
        send_a2[rows0, :] = recv_a1[0, rows0] + vx_fa[rows0, :]
        a2a.start()
        b1_0a.wait_recv()
        cp_fb.wait()
        send_b2[rows0, :] = recv_b1[0, rows0] + vx_fb[rows0, :]
        b2a.start()
        a1_0b.wait_recv()
        send_a2[rows1, :] = recv_a1[0, rows1] + vx_fa[rows1, :]
        a2b.start()
        b1_0b.wait_recv()
        send_b2[rows1, :] = recv_b1[0, rows1] + vx_fb[rows1, :]
        b2b.start()

        a1_1.wait_recv()
        cp_own.wait()
        recv_a1[0, :, :] = recv_a1[1] + vx_own[:, cols_a]
        b1_1.wait_recv()
        recv_b1[0, :, :] = recv_b1[1] + vx_own[:, cols_b]
        cp_g.wait()

        for h, rws in enumerate((rows0, rows1)):
            (a2a, a2b)[h].wait_recv()
            y_a = recv_a1[0, rws] + recv_a2[rws, :]
            (b2a, b2b)[h].wait_recv()
            y_b = recv_b1[0, rws] + recv_b2[rws, :]
            ssq = (jnp.sum(y_a * y_a, axis=-1, keepdims=True)
                   + jnp.sum(y_b * y_b, axis=-1, keepdims=True))
            scale = lax.rsqrt(ssq / D + 1e-6)
            out_ref[rws, cols_a] = y_a * scale * vg[:, cols_a]
            out_ref[rws, cols_b] = y_b * scale * vg[:, cols_b]

        for d in (a1_0a, a1_0b, a1_1, b1_0a, b1_0b, b1_1,
                  a2a, a2b, b2a, b2b):
            d.wait_send()

    return pl.pallas_call(
        body,
        out_shape=jax.ShapeDtypeStruct((M_PER, D), jnp.float32),
        in_specs=[
            pl.BlockSpec(memory_space=pl.ANY),
            pl.BlockSpec(memory_space=pl.ANY),
        ],
        out_specs=pl.BlockSpec(memory_space=pltpu.VMEM),
        scratch_shapes=[
            pltpu.VMEM((2, M_PER, DH), jnp.float32),
            pltpu.VMEM((2, M_PER, DH), jnp.float32),
            pltpu.VMEM((M_PER, DH), jnp.float32),
            pltpu.VMEM((M_PER, DH), jnp.float32),
            pltpu.VMEM((M_PER, DH), jnp.float32),
            pltpu.VMEM((M_PER, DH), jnp.float32),
            pltpu.VMEM((M_PER, D), jnp.float32),
            pltpu.VMEM((M_PER, DH), jnp.float32),
            pltpu.VMEM((M_PER, DH), jnp.float32),
            pltpu.VMEM((1, D), jnp.float32),
            pltpu.SemaphoreType.DMA((5,)),
            pltpu.SemaphoreType.DMA((5,)),
            pltpu.SemaphoreType.DMA((5,)),
            pltpu.SemaphoreType.DMA((5,)),
            pltpu.SemaphoreType.DMA((4,)),
        ],
        compiler_params=pltpu.CompilerParams(collective_id=0),
    )(x, g)


# device time: 23299 ns/iter; 1.0030x vs baseline; 1.0007x over previous
import jax
import jax.numpy as jnp
from jax import lax
from jax.experimental import pallas as pl
from jax.experimental.pallas import tpu as pltpu

N_DEV = 4
M_PER = 512
HM = M_PER // 2
D = 512
DH = D // 2


def kernel(partial, gamma):
    x = partial.reshape(N_DEV * M_PER, D)
    g = gamma.reshape(1, D)
    x = pltpu.with_memory_space_constraint(x, pltpu.MemorySpace.HBM)
    g = pltpu.with_memory_space_constraint(g, pltpu.MemorySpace.HBM)

    def body(x_ref, g_ref, out_ref,
             recv_a1, recv_b1, send_a2, send_b2, recv_a2, recv_b2,
             vx_own, vx_fa, vx_fb, vg,
             sems_send_a, sems_recv_a, sems_send_b, sems_recv_b,
             sems_local):
        my = lax.axis_index("i")
        left = lax.rem(my + N_DEV - 1, N_DEV)
        right = lax.rem(my + 1, N_DEV)
        q = my ^ 1
        r = 3 - my

        def row(c, h=None):
            if h is None:
                return pl.ds(c * M_PER, M_PER)
            return pl.ds(c * M_PER + h * HM, HM)

        rows0 = pl.ds(0, HM)
        rows1 = pl.ds(HM, HM)
        cols_a = pl.ds(0, DH)
        cols_b = pl.ds(DH, DH)

        cp_own = pltpu.make_async_copy(
            x_ref.at[row(my), :], vx_own, sems_local.at[0])
        cp_fa = pltpu.make_async_copy(
            x_ref.at[row(3 - my), cols_a], vx_fa, sems_local.at[1])
        cp_fb = pltpu.make_async_copy(
            x_ref.at[row(q), cols_b], vx_fb, sems_local.at[2])
        cp_g = pltpu.make_async_copy(g_ref, vg, sems_local.at[3])
        cp_fa.start()
        cp_fb.start()
        cp_own.start()
        cp_g.start()

        barrier_sem = pltpu.get_barrier_semaphore()
        for nbr in (left, right):
            pl.semaphore_signal(
                barrier_sem, inc=1,
                device_id=(nbr,), device_id_type=pl.DeviceIdType.MESH,
            )
        pl.semaphore_wait(barrier_sem, 2)

        def remote(src, dst, ssem, rsem, dev):
            return pltpu.make_async_remote_copy(
                src_ref=src, dst_ref=dst, send_sem=ssem, recv_sem=rsem,
                device_id=(dev,), device_id_type=pl.DeviceIdType.MESH,
            )

        a1_0a = remote(x_ref.at[row(3 - q, 0), cols_a],
                       recv_a1.at[0, rows0], sems_send_a.at[0],
                       sems_recv_a.at[0], q)
        a1_0b = remote(x_ref.at[row(3 - q, 1), cols_a],
                       recv_a1.at[0, rows1], sems_send_a.at[1],
                       sems_recv_a.at[1], q)
        a1_1 = remote(x_ref.at[row(q), cols_a],
                      recv_a1.at[1], sems_send_a.at[2],
                      sems_recv_a.at[2], q)
        b1_0a = remote(x_ref.at[row(r ^ 1, 0), cols_b],
                       recv_b1.at[0, rows0], sems_send_b.at[0],
                       sems_recv_b.at[0], r)
        b1_0b = remote(x_ref.at[row(r ^ 1, 1), cols_b],
                       recv_b1.at[0, rows1], sems_send_b.at[1],
                       sems_recv_b.at[1], r)
        b1_1 = remote(x_ref.at[row(r), cols_b],
                      recv_b1.at[1], sems_send_b.at[2],
                      sems_recv_b.at[2], r)
        a1_0a.start()
        b1_0a.start()
        a1_0b.start()
        b1_0b.start()
        a1_1.start()
        b1_1.start()

        QR = M_PER // 4
        rq = [pl.ds(h * QR, QR) for h in range(4)]
        a2 = [remote(send_a2.at[rq[h]], recv_a2.at[rq[h]],
                     sems_send_a.at[3 + h], sems_recv_a.at[3 + h], r)
              for h in range(4)]
        b2 = [remote(send_b2.at[rq[h]], recv_b2.at[rq[h]],
                     sems_send_b.at[3 + h], sems_recv_b.at[3 + h], q)
              for h in range(4)]

        a1_0a.wait_recv()
        cp_fa.wait()
        send_a2[rows0, :] = recv_a1[0, rows0] + vx_fa[rows0, :]
        a2[0].start()
        a2[1].start()
        b1_0a.wait_recv()
        cp_fb.wait()
        send_b2[rows0, :] = recv_b1[0, rows0] + vx_fb[rows0, :]
        b2[0].start()
        b2[1].start()
        a1_0b.wait_recv()
        send_a2[rows1, :] = recv_a1[0, rows1] + vx_fa[rows1, :]
        a2[2].start()
        a2[3].start()
        b1_0b.wait_recv()
        send_b2[rows1, :] = recv_b1[0, rows1] + vx_fb[rows1, :]
        b2[2].start()
        b2[3].start()

        a1_1.wait_recv()
        cp_own.wait()
        recv_a1[0, :, :] = recv_a1[1] + vx_own[:, cols_a]
        b1_1.wait_recv()
        recv_b1[0, :, :] = recv_b1[1] + vx_own[:, cols_b]
        cp_g.wait()

        for h in range(4):
            rws = rq[h]
            a2[h].wait_recv()
            y_a = recv_a1[0, rws] + recv_a2[rws, :]
            b2[h].wait_recv()
            y_b = recv_b1[0, rws] + recv_b2[rws, :]
            ssq = (jnp.sum(y_a * y_a, axis=-1, keepdims=True)
                   + jnp.sum(y_b * y_b, axis=-1, keepdims=True))
            scale = lax.rsqrt(ssq / D + 1e-6)
            out_ref[rws, cols_a] = y_a * scale * vg[:, cols_a]
            out_ref[rws, cols_b] = y_b * scale * vg[:, cols_b]

        for d in (a1_0a, a1_0b, a1_1, b1_0a, b1_0b, b1_1, *a2, *b2):
            d.wait_send()

    return pl.pallas_call(
        body,
        out_shape=jax.ShapeDtypeStruct((M_PER, D), jnp.float32),
        in_specs=[
            pl.BlockSpec(memory_space=pl.ANY),
            pl.BlockSpec(memory_space=pl.ANY),
        ],
        out_specs=pl.BlockSpec(memory_space=pltpu.VMEM),
        scratch_shapes=[
            pltpu.VMEM((2, M_PER, DH), jnp.float32),
            pltpu.VMEM((2, M_PER, DH), jnp.float32),
            pltpu.VMEM((M_PER, DH), jnp.float32),
            pltpu.VMEM((M_PER, DH), jnp.float32),
            pltpu.VMEM((M_PER, DH), jnp.float32),
            pltpu.VMEM((M_PER, DH), jnp.float32),
            pltpu.VMEM((M_PER, D), jnp.float32),
            pltpu.VMEM((M_PER, DH), jnp.float32),
            pltpu.VMEM((M_PER, DH), jnp.float32),
            pltpu.VMEM((1, D), jnp.float32),
            pltpu.SemaphoreType.DMA((7,)),
            pltpu.SemaphoreType.DMA((7,)),
            pltpu.SemaphoreType.DMA((7,)),
            pltpu.SemaphoreType.DMA((7,)),
            pltpu.SemaphoreType.DMA((4,)),
        ],
        compiler_params=pltpu.CompilerParams(collective_id=0),
    )(x, g)
